# baseline (speedup 1.0000x reference)
import jax
import jax.numpy as jnp
from jax import lax
from jax.experimental import pallas as pl
from jax.experimental.pallas import tpu as pltpu
from jax.experimental.pallas import tpu_sc as plsc

PH, PW = 16, 64
NQ = PH * PW
SCALE = 0.25
N, C, H, W = 2, 128, 256, 256
R = 256
RB = 32

NW = 32
ROIS_PER_W = R // NW
QT = NQ // 4
CPTS = 32
NCHUNK = QT // CPTS
NT = 4 * ROIS_PER_W


def _coord_body(bez_ref, idx_ref, wts_ref):
    bez = bez_ref[...]
    b = bez[:, 0].astype(jnp.int32)
    p = bez[:, 1:17] * SCALE
    q = lax.broadcasted_iota(jnp.int32, (RB, NQ), 1)
    u = (q & (PW - 1)).astype(jnp.float32) * (1.0 / PW)
    v = (q >> 6).astype(jnp.float32) * (1.0 / PH)

    def bez3(a0, a1, a2, a3, t):
        return ((1.0 - t) ** 3) * a0 + 3.0 * t * ((1.0 - t) ** 2) * a1 \
            + 3.0 * (t ** 2) * (1.0 - t) * a2 + (t ** 3) * a3

    x0 = bez3(p[:, 0:1], p[:, 2:3], p[:, 4:5], p[:, 6:7], u)
    y0 = bez3(p[:, 1:2], p[:, 3:4], p[:, 5:6], p[:, 7:8], u)
    x1 = bez3(p[:, 8:9], p[:, 10:11], p[:, 12:13], p[:, 14:15], u)
    y1 = bez3(p[:, 9:10], p[:, 11:12], p[:, 13:14], p[:, 15:16], u)
    x_c = x1 * v + x0 * (1.0 - v)
    y_c = y1 * v + y0 * (1.0 - v)
    invalid = (y_c < -1.0) | (y_c > H) | (x_c < -1.0) | (x_c > W)
    y = jnp.maximum(y_c, 0.0)
    x = jnp.maximum(x_c, 0.0)
    y_low = jnp.floor(y).astype(jnp.int32)
    x_low = jnp.floor(x).astype(jnp.int32)
    y_cond = y_low >= H - 1
    x_cond = x_low >= W - 1
    y_high = jnp.where(y_cond, H - 1, y_low + 1)
    y_low = jnp.where(y_cond, H - 1, y_low)
    y = jnp.where(y_cond, y_low.astype(jnp.float32), y)
    x_high = jnp.where(x_cond, W - 1, x_low + 1)
    x_low = jnp.where(x_cond, W - 1, x_low)
    x = jnp.where(x_cond, x_low.astype(jnp.float32), x)
    ly = y - y_low.astype(jnp.float32)
    lx = x - x_low.astype(jnp.float32)
    hy = 1.0 - ly
    hx = 1.0 - lx
    yl = jnp.clip(y_low, 0, H - 1)
    yh = jnp.clip(y_high, 0, H - 1)
    xl = jnp.clip(x_low, 0, W - 1)
    xh = jnp.clip(x_high, 0, W - 1)
    bq = b[:, None] * (H * W)
    pix = [bq + yl * W + xl, bq + yl * W + xh,
           bq + yh * W + xl, bq + yh * W + xh]
    idx_ref[...] = jnp.concatenate(
        [p_[:, qq * QT + k * CPTS:qq * QT + (k + 1) * CPTS]
         for qq in range(4) for k in range(NCHUNK) for p_ in pix], axis=1)
    zero = jnp.zeros_like(hy)
    wts = [jnp.where(invalid, zero, w)
           for w in (hy * hx, hy * lx, ly * hx, ly * lx)]
    wts_ref[...] = jnp.concatenate(
        [w[:, qq * QT:(qq + 1) * QT] for qq in range(4) for w in wts], axis=1)


def _coords(beziers):
    idx, wts = pl.pallas_call(
        _coord_body,
        grid=(R // RB,),
        in_specs=[pl.BlockSpec((RB, 17), lambda i: (i, 0))],
        out_specs=[
            pl.BlockSpec((RB, 4 * NQ), lambda i: (i, 0)),
            pl.BlockSpec((RB, 4 * NQ), lambda i: (i, 0)),
        ],
        out_shape=[
            jax.ShapeDtypeStruct((R, 4 * NQ), jnp.int32),
            jax.ShapeDtypeStruct((R, 4 * NQ), jnp.float32),
        ],
    )(beziers)
    return idx, wts


def _sc_body(table_ref, idx_hbm, wts_hbm, out_hbm,
             idx_v, wts_v, rows_v, acc_v,
             semg0, semg1, semg2, semo0, semo1, semst0, semst1):
    cid = lax.axis_index("c")
    sid = lax.axis_index("s")
    wid = sid * 2 + cid
    iota16 = lax.iota(jnp.int32, 16)
    semg = (semg0, semg1, semg2)
    semo = (semo0, semo1)
    semst = (semst0, semst1)

    def rq(t):
        return wid * ROIS_PER_W + (t >> 2), t & 3

    def stage_start(t, sb):
        r, qq = rq(t)
        pltpu.async_copy(idx_hbm.at[r, qq], idx_v.at[sb], semst[sb])
        pltpu.async_copy(wts_hbm.at[r, qq], wts_v.at[sb], semst[sb])

    def stage_wait(t, sb):
        r, qq = rq(t)
        pltpu.make_async_copy(idx_hbm.at[r, qq], idx_v.at[sb],
                              semst[sb]).wait()
        pltpu.make_async_copy(wts_hbm.at[r, qq], wts_v.at[sb],
                              semst[sb]).wait()

    def out_start(t, sb):
        r, qq = rq(t)
        pltpu.async_copy(acc_v.at[sb],
                         out_hbm.at[r, :, pl.ds(qq * QT, QT)], semo[sb])

    def out_wait(t, sb):
        r, qq = rq(t)
        pltpu.make_async_copy(acc_v.at[sb],
                              out_hbm.at[r, :, pl.ds(qq * QT, QT)],
                              semo[sb]).wait()

    def start(k, b, sb):
        for s in range(2):
            pltpu.async_copy(
                table_ref.at[idx_v.at[sb, k, s]],
                rows_v.at[b, pl.ds(s * 2 * CPTS, 2 * CPTS)], semg[b])

    def wait(k, b, sb):
        for s in range(2):
            pltpu.make_async_copy(
                table_ref.at[idx_v.at[sb, k, s]],
                rows_v.at[b, pl.ds(s * 2 * CPTS, 2 * CPTS)],
                semg[b]).wait()

    def compute(k, b, sb):
        for g in range(CPTS // 16):
            qb = k * CPTS + g * 16
            w0 = wts_v[sb, 0, pl.ds(qb, 16)]
            w1 = wts_v[sb, 1, pl.ds(qb, 16)]
            w2 = wts_v[sb, 2, pl.ds(qb, 16)]
            w3 = wts_v[sb, 3, pl.ds(qb, 16)]
            p0 = iota16 + (0 * CPTS + g * 16)
            p1 = iota16 + (1 * CPTS + g * 16)
            p2 = iota16 + (2 * CPTS + g * 16)
            p3 = iota16 + (3 * CPTS + g * 16)
            qcol = iota16 + qb

            def ch_body(c, chv):
                chvb = (chv + C // 4) & (C - 1)
                chvc = (chv + 2 * C // 4) & (C - 1)
                chvd = (chv + 3 * C // 4) & (C - 1)
                a0 = plsc.load_gather(rows_v.at[b], [p0, chv])
                a1 = plsc.load_gather(rows_v.at[b], [p1, chv])
                a2 = plsc.load_gather(rows_v.at[b], [p2, chv])
                a3 = plsc.load_gather(rows_v.at[b], [p3, chv])
                b0 = plsc.load_gather(rows_v.at[b], [p0, chvb])
                b1 = plsc.load_gather(rows_v.at[b], [p1, chvb])
                b2 = plsc.load_gather(rows_v.at[b], [p2, chvb])
                b3 = plsc.load_gather(rows_v.at[b], [p3, chvb])
                c0 = plsc.load_gather(rows_v.at[b], [p0, chvc])
                c1 = plsc.load_gather(rows_v.at[b], [p1, chvc])
                c2 = plsc.load_gather(rows_v.at[b], [p2, chvc])
                c3 = plsc.load_gather(rows_v.at[b], [p3, chvc])
                d0 = plsc.load_gather(rows_v.at[b], [p0, chvd])
                d1 = plsc.load_gather(rows_v.at[b], [p1, chvd])
                d2 = plsc.load_gather(rows_v.at[b], [p2, chvd])
                d3 = plsc.load_gather(rows_v.at[b], [p3, chvd])
                va = (w0 * a0 + w1 * a1) + (w2 * a2 + w3 * a3)
                vb = (w0 * b0 + w1 * b1) + (w2 * b2 + w3 * b3)
                vc = (w0 * c0 + w1 * c1) + (w2 * c2 + w3 * c3)
                vd = (w0 * d0 + w1 * d1) + (w2 * d2 + w3 * d3)
                plsc.store_scatter(acc_v.at[sb], [chv, qcol], va)
                plsc.store_scatter(acc_v.at[sb], [chvb, qcol], vb)
                plsc.store_scatter(acc_v.at[sb], [chvc, qcol], vc)
                plsc.store_scatter(acc_v.at[sb], [chvd, qcol], vd)
                return (chv + 1) & (C - 1)

            lax.fori_loop(0, C // 4, ch_body, iota16, unroll=4)

    def one_task(t, sb):
        stage_wait(t, sb)

        @pl.when(t < NT - 1)
        def _():
            stage_start(t + 1, 1 - sb)

        @pl.when(t >= 2)
        def _():
            out_wait(t - 2, sb)

        start(0, 0, sb)
        start(1, 1, sb)
        for k in range(NCHUNK):
            b = k % 3
            if k + 2 < NCHUNK:
                start(k + 2, (k + 2) % 3, sb)
            wait(k, b, sb)
            compute(k, b, sb)
        out_start(t, sb)

    def task_pair(tp, carry0):
        one_task(2 * tp, 0)
        one_task(2 * tp + 1, 1)
        return carry0

    stage_start(0, 0)
    lax.fori_loop(0, NT // 2, task_pair, 0)
    out_wait(NT - 2, 0)
    out_wait(NT - 1, 1)


def _sc_run(table, idx, wts):
    run = pl.kernel(
        _sc_body,
        out_type=jax.ShapeDtypeStruct((R, C, NQ), jnp.float32),
        mesh=plsc.VectorSubcoreMesh(core_axis_name="c", subcore_axis_name="s"),
        compiler_params=pltpu.CompilerParams(needs_layout_passes=False),
        scratch_types=[
            pltpu.VMEM((2, NCHUNK, 2, 2 * CPTS), jnp.int32),
            pltpu.VMEM((2, 4, QT), jnp.float32),
            pltpu.VMEM((3, 4 * CPTS, C), jnp.float32),
            pltpu.VMEM((2, C, QT), jnp.float32),
            pltpu.SemaphoreType.DMA,
            pltpu.SemaphoreType.DMA,
            pltpu.SemaphoreType.DMA,
            pltpu.SemaphoreType.DMA,
            pltpu.SemaphoreType.DMA,
            pltpu.SemaphoreType.DMA,
            pltpu.SemaphoreType.DMA,
        ],
    )
    return run(table, idx, wts)


def kernel(input, beziers):
    table = jnp.transpose(input, (0, 2, 3, 1)).reshape(N * H * W, C)
    idx, wts = _coords(beziers)
    idx = idx.reshape(R, 4, NCHUNK, 2, 2 * CPTS)
    wts = wts.reshape(R, 4, 4, QT)
    out = _sc_run(table, idx, wts)
    return out.reshape(R, C, PH, PW)


# speedup vs baseline: 1.0197x; 1.0197x over previous
import jax
import jax.numpy as jnp
from jax import lax
from jax.experimental import pallas as pl
from jax.experimental.pallas import tpu as pltpu
from jax.experimental.pallas import tpu_sc as plsc

PH, PW = 16, 64
NQ = PH * PW
SCALE = 0.25
N, C, H, W = 2, 128, 256, 256
R = 256
RB = 32

NW = 32
ROIS_PER_W = R // NW
QT = NQ // 4
CPTS = 32
NCHUNK = QT // CPTS
NT = 4 * ROIS_PER_W


def _coord_body(bez_ref, idx_ref, wts_ref):
    bez = bez_ref[...]
    b = bez[:, 0].astype(jnp.int32)
    p = bez[:, 1:17] * SCALE
    q = lax.broadcasted_iota(jnp.int32, (RB, NQ), 1)
    u = (q & (PW - 1)).astype(jnp.float32) * (1.0 / PW)
    v = (q >> 6).astype(jnp.float32) * (1.0 / PH)

    def bez3(a0, a1, a2, a3, t):
        return ((1.0 - t) ** 3) * a0 + 3.0 * t * ((1.0 - t) ** 2) * a1 \
            + 3.0 * (t ** 2) * (1.0 - t) * a2 + (t ** 3) * a3

    x0 = bez3(p[:, 0:1], p[:, 2:3], p[:, 4:5], p[:, 6:7], u)
    y0 = bez3(p[:, 1:2], p[:, 3:4], p[:, 5:6], p[:, 7:8], u)
    x1 = bez3(p[:, 8:9], p[:, 10:11], p[:, 12:13], p[:, 14:15], u)
    y1 = bez3(p[:, 9:10], p[:, 11:12], p[:, 13:14], p[:, 15:16], u)
    x_c = x1 * v + x0 * (1.0 - v)
    y_c = y1 * v + y0 * (1.0 - v)
    invalid = (y_c < -1.0) | (y_c > H) | (x_c < -1.0) | (x_c > W)
    y = jnp.maximum(y_c, 0.0)
    x = jnp.maximum(x_c, 0.0)
    y_low = jnp.floor(y).astype(jnp.int32)
    x_low = jnp.floor(x).astype(jnp.int32)
    y_cond = y_low >= H - 1
    x_cond = x_low >= W - 1
    y_high = jnp.where(y_cond, H - 1, y_low + 1)
    y_low = jnp.where(y_cond, H - 1, y_low)
    y = jnp.where(y_cond, y_low.astype(jnp.float32), y)
    x_high = jnp.where(x_cond, W - 1, x_low + 1)
    x_low = jnp.where(x_cond, W - 1, x_low)
    x = jnp.where(x_cond, x_low.astype(jnp.float32), x)
    ly = y - y_low.astype(jnp.float32)
    lx = x - x_low.astype(jnp.float32)
    hy = 1.0 - ly
    hx = 1.0 - lx
    yl = jnp.clip(y_low, 0, H - 1)
    yh = jnp.clip(y_high, 0, H - 1)
    xl = jnp.clip(x_low, 0, W - 1)
    xh = jnp.clip(x_high, 0, W - 1)
    bq = b[:, None] * (H * W)
    pix = [bq + yl * W + xl, bq + yl * W + xh,
           bq + yh * W + xl, bq + yh * W + xh]
    idx_ref[...] = jnp.concatenate(
        [p_[:, qq * QT + k * CPTS:qq * QT + (k + 1) * CPTS]
         for qq in range(4) for k in range(NCHUNK) for p_ in pix], axis=1)
    zero = jnp.zeros_like(hy)
    wts = [jnp.where(invalid, zero, w)
           for w in (hy * hx, hy * lx, ly * hx, ly * lx)]
    wts_ref[...] = jnp.concatenate(
        [w[:, qq * QT:(qq + 1) * QT] for qq in range(4) for w in wts], axis=1)


def _coords(beziers):
    idx, wts = pl.pallas_call(
        _coord_body,
        grid=(R // RB,),
        in_specs=[pl.BlockSpec((RB, 17), lambda i: (i, 0))],
        out_specs=[
            pl.BlockSpec((RB, 4 * NQ), lambda i: (i, 0)),
            pl.BlockSpec((RB, 4 * NQ), lambda i: (i, 0)),
        ],
        out_shape=[
            jax.ShapeDtypeStruct((R, 4 * NQ), jnp.int32),
            jax.ShapeDtypeStruct((R, 4 * NQ), jnp.float32),
        ],
    )(beziers)
    return idx, wts


def _sc_body(table_ref, idx_hbm, wts_hbm, out_hbm,
             idx_v, wts_v, rows_v, acc_v,
             semg0, semg1, semg2, semo0, semo1, semst0, semst1):
    cid = lax.axis_index("c")
    sid = lax.axis_index("s")
    wid = sid * 2 + cid
    iota16 = lax.iota(jnp.int32, 16)
    semg = (semg0, semg1, semg2)
    semo = (semo0, semo1)
    semst = (semst0, semst1)

    def rq(t):
        return wid * ROIS_PER_W + (t >> 2), t & 3

    def stage_start(t, sb):
        r, qq = rq(t)
        pltpu.async_copy(idx_hbm.at[r, qq], idx_v.at[sb], semst[sb])
        pltpu.async_copy(wts_hbm.at[r, qq], wts_v.at[sb], semst[sb])

    def stage_wait(t, sb):
        r, qq = rq(t)
        pltpu.make_async_copy(idx_hbm.at[r, qq], idx_v.at[sb],
                              semst[sb]).wait()
        pltpu.make_async_copy(wts_hbm.at[r, qq], wts_v.at[sb],
                              semst[sb]).wait()

    def out_start(t, sb):
        r, qq = rq(t)
        pltpu.async_copy(acc_v.at[sb],
                         out_hbm.at[r, :, pl.ds(qq * QT, QT)], semo[sb])

    def out_wait(t, sb):
        r, qq = rq(t)
        pltpu.make_async_copy(acc_v.at[sb],
                              out_hbm.at[r, :, pl.ds(qq * QT, QT)],
                              semo[sb]).wait()

    def start(k, b, sb):
        for s in range(2):
            pltpu.async_copy(
                table_ref.at[idx_v.at[sb, k, s]],
                rows_v.at[b, pl.ds(s * 2 * CPTS, 2 * CPTS)], semg[b])

    def wait(k, b, sb):
        for s in range(2):
            pltpu.make_async_copy(
                table_ref.at[idx_v.at[sb, k, s]],
                rows_v.at[b, pl.ds(s * 2 * CPTS, 2 * CPTS)],
                semg[b]).wait()

    def compute(k, b, sb):
        for g in range(CPTS // 16):
            qb = k * CPTS + g * 16
            w0 = wts_v[sb, 0, pl.ds(qb, 16)]
            w1 = wts_v[sb, 1, pl.ds(qb, 16)]
            w2 = wts_v[sb, 2, pl.ds(qb, 16)]
            w3 = wts_v[sb, 3, pl.ds(qb, 16)]
            p0 = iota16 + (0 * CPTS + g * 16)
            p1 = iota16 + (1 * CPTS + g * 16)
            p2 = iota16 + (2 * CPTS + g * 16)
            p3 = iota16 + (3 * CPTS + g * 16)
            qcol = iota16 + qb

            def ch_body(c, chv):
                chvb = (chv + C // 4) & (C - 1)
                chvc = (chv + 2 * C // 4) & (C - 1)
                chvd = (chv + 3 * C // 4) & (C - 1)
                a0 = plsc.load_gather(rows_v.at[b], [p0, chv])
                a1 = plsc.load_gather(rows_v.at[b], [p1, chv])
                a2 = plsc.load_gather(rows_v.at[b], [p2, chv])
                a3 = plsc.load_gather(rows_v.at[b], [p3, chv])
                b0 = plsc.load_gather(rows_v.at[b], [p0, chvb])
                b1 = plsc.load_gather(rows_v.at[b], [p1, chvb])
                b2 = plsc.load_gather(rows_v.at[b], [p2, chvb])
                b3 = plsc.load_gather(rows_v.at[b], [p3, chvb])
                c0 = plsc.load_gather(rows_v.at[b], [p0, chvc])
                c1 = plsc.load_gather(rows_v.at[b], [p1, chvc])
                c2 = plsc.load_gather(rows_v.at[b], [p2, chvc])
                c3 = plsc.load_gather(rows_v.at[b], [p3, chvc])
                d0 = plsc.load_gather(rows_v.at[b], [p0, chvd])
                d1 = plsc.load_gather(rows_v.at[b], [p1, chvd])
                d2 = plsc.load_gather(rows_v.at[b], [p2, chvd])
                d3 = plsc.load_gather(rows_v.at[b], [p3, chvd])
                va = (w0 * a0 + w1 * a1) + (w2 * a2 + w3 * a3)
                vb = (w0 * b0 + w1 * b1) + (w2 * b2 + w3 * b3)
                vc = (w0 * c0 + w1 * c1) + (w2 * c2 + w3 * c3)
                vd = (w0 * d0 + w1 * d1) + (w2 * d2 + w3 * d3)
                plsc.store_scatter(acc_v.at[sb], [chv, qcol], va)
                plsc.store_scatter(acc_v.at[sb], [chvb, qcol], vb)
                plsc.store_scatter(acc_v.at[sb], [chvc, qcol], vc)
                plsc.store_scatter(acc_v.at[sb], [chvd, qcol], vd)
                return (chv + 1) & (C - 1)

            lax.fori_loop(0, C // 4, ch_body, iota16, unroll=2)

    def one_task(t, sb):
        stage_wait(t, sb)

        @pl.when(t < NT - 1)
        def _():
            stage_start(t + 1, 1 - sb)

        @pl.when(t >= 2)
        def _():
            out_wait(t - 2, sb)

        start(0, 0, sb)
        start(1, 1, sb)
        for k in range(NCHUNK):
            b = k % 3
            if k + 2 < NCHUNK:
                start(k + 2, (k + 2) % 3, sb)
            wait(k, b, sb)
            compute(k, b, sb)
        out_start(t, sb)

    def task_pair(tp, carry0):
        one_task(2 * tp, 0)
        one_task(2 * tp + 1, 1)
        return carry0

    stage_start(0, 0)
    lax.fori_loop(0, NT // 2, task_pair, 0)
    out_wait(NT - 2, 0)
    out_wait(NT - 1, 1)


def _sc_run(table, idx, wts):
    run = pl.kernel(
        _sc_body,
        out_type=jax.ShapeDtypeStruct((R, C, NQ), jnp.float32),
        mesh=plsc.VectorSubcoreMesh(core_axis_name="c", subcore_axis_name="s"),
        compiler_params=pltpu.CompilerParams(needs_layout_passes=False),
        scratch_types=[
            pltpu.VMEM((2, NCHUNK, 2, 2 * CPTS), jnp.int32),
            pltpu.VMEM((2, 4, QT), jnp.float32),
            pltpu.VMEM((3, 4 * CPTS, C), jnp.float32),
            pltpu.VMEM((2, C, QT), jnp.float32),
            pltpu.SemaphoreType.DMA,
            pltpu.SemaphoreType.DMA,
            pltpu.SemaphoreType.DMA,
            pltpu.SemaphoreType.DMA,
            pltpu.SemaphoreType.DMA,
            pltpu.SemaphoreType.DMA,
            pltpu.SemaphoreType.DMA,
        ],
    )
    return run(table, idx, wts)


def kernel(input, beziers):
    table = jnp.transpose(input, (0, 2, 3, 1)).reshape(N * H * W, C)
    idx, wts = _coords(beziers)
    idx = idx.reshape(R, 4, NCHUNK, 2, 2 * CPTS)
    wts = wts.reshape(R, 4, 4, QT)
    out = _sc_run(table, idx, wts)
    return out.reshape(R, C, PH, PW)
